# Initial kernel scaffold; baseline (speedup 1.0000x reference)
import jax
import jax.numpy as jnp
from jax.experimental import pallas as pl


def kernel(points_ref, points):
    raise NotImplementedError("write your pallas kernel here")


# speedup vs baseline: 18.0390x; 18.0390x over previous
import functools

import numpy as np
import jax
import jax.numpy as jnp
from jax import lax
from jax.experimental import pallas as pl
from jax.experimental.pallas import tpu as pltpu
from jax.experimental.pallas import tpu_sc as plsc

_B = 4
_N = 4096
_K = 16
_L = 16
_NBLK = _N // _L
_NC = 2
_NS = 16
_NW = _NC * _NS
_WPB = _NW // _B
_ROWS = _N // _WPB
_BIG = np.float32(3.0e38)


def _sqrt16(a):
    i = plsc.bitcast(a, jnp.int32)
    i = jnp.int32(0x5F3759DF) - (i >> 1)
    y = plsc.bitcast(i, jnp.float32)
    ah = a * jnp.float32(0.5)
    y = y * (jnp.float32(1.5) - ah * y * y)
    y = y * (jnp.float32(1.5) - ah * y * y)
    y = y * (jnp.float32(1.5) - ah * y * y)
    return jnp.where(a > 0.0, a * y, jnp.float32(0.0))


def _body(rx_hbm, ry_hbm, rz_hbm, px_hbm, py_hbm, pz_hbm, out_hbm,
          xs, ys, zs, pxs, pys, pzs, accv):
    wid = lax.axis_index("s") * _NC + lax.axis_index("c")
    batch = wid // _WPB
    row0 = (wid % _WPB) * _ROWS

    boff = batch * _N
    pltpu.sync_copy(rx_hbm.at[pl.ds(boff, _N)], xs)
    pltpu.sync_copy(ry_hbm.at[pl.ds(boff, _N)], ys)
    pltpu.sync_copy(rz_hbm.at[pl.ds(boff, _N)], zs)
    pltpu.sync_copy(px_hbm.at[pl.ds(boff, _N)], pxs)
    pltpu.sync_copy(py_hbm.at[pl.ds(boff, _N)], pys)
    pltpu.sync_copy(pz_hbm.at[pl.ds(boff, _N)], pzs)

    iota = lax.iota(jnp.int32, _L)

    def row_body(i, acc_comp):
        acc, comp = acc_comp
        r = row0 + i
        rv = jnp.full((_L,), r, jnp.int32)
        qx = plsc.load_gather(xs, [rv])
        qy = plsc.load_gather(ys, [rv])
        qz = plsc.load_gather(zs, [rv])

        def cand_body(c, carry):
            bk, bv = carry
            base = c * _L
            xv = xs[pl.ds(base, _L)]
            yv = ys[pl.ds(base, _L)]
            zv = zs[pl.ds(base, _L)]
            dx = xv - qx
            dy = yv - qy
            dz = zv - qz
            d2 = dx * dx + dy * dy + dz * dz
            idxv = iota + base
            d2 = jnp.where(idxv == rv, _BIG, d2)
            sk, sv = plsc.sort_key_val(d2, idxv)
            rk = lax.rev(sk, (0,))
            rsv = lax.rev(sv, (0,))
            take = bk <= rk
            mk = jnp.where(take, bk, rk)
            mv = jnp.where(take, bv, rsv)
            bk, bv = plsc.sort_key_val(mk, mv)
            return bk, bv

        bk0 = jnp.full((_L,), _BIG, jnp.float32)
        bv0 = jnp.zeros((_L,), jnp.int32)
        bk, bv = lax.fori_loop(0, _NBLK, cand_body, (bk0, bv0))

        dist_ref = _sqrt16(bk)

        qpx = plsc.load_gather(pxs, [rv])
        qpy = plsc.load_gather(pys, [rv])
        qpz = plsc.load_gather(pzs, [rv])
        nx = plsc.load_gather(pxs, [bv])
        ny = plsc.load_gather(pys, [bv])
        nz = plsc.load_gather(pzs, [bv])
        ddx = nx - qpx
        ddy = ny - qpy
        ddz = nz - qpz
        dist = _sqrt16(ddx * ddx + ddy * ddy + ddz * ddz)
        y = jnp.abs(dist_ref - dist) - comp
        t = acc + y
        comp = (t - acc) - y
        return t, comp

    zero = jnp.zeros((_L,), jnp.float32)
    acc, _ = lax.fori_loop(0, _ROWS, row_body, (zero, zero))
    accv[...] = acc
    pltpu.sync_copy(accv, out_hbm.at[wid])


@jax.jit
def _partials(rx, ry, rz, px, py, pz):
    mesh = plsc.VectorSubcoreMesh(
        core_axis_name="c", subcore_axis_name="s",
        num_cores=_NC, num_subcores=_NS)
    f = pl.kernel(
        _body,
        out_type=jax.ShapeDtypeStruct((_NW, _L), jnp.float32),
        mesh=mesh,
        scratch_types=[
            pltpu.VMEM((_N,), jnp.float32),
            pltpu.VMEM((_N,), jnp.float32),
            pltpu.VMEM((_N,), jnp.float32),
            pltpu.VMEM((_N,), jnp.float32),
            pltpu.VMEM((_N,), jnp.float32),
            pltpu.VMEM((_N,), jnp.float32),
            pltpu.VMEM((_L,), jnp.float32),
        ],
        compiler_params=pltpu.CompilerParams(needs_layout_passes=False),
    )
    return f(rx, ry, rz, px, py, pz)


def kernel(points_ref, points):
    rx, ry, rz = (points_ref[:, :, i].reshape(-1) for i in range(3))
    px, py, pz = (points[:, :, i].reshape(-1) for i in range(3))
    partials = _partials(rx, ry, rz, px, py, pz)
    return jnp.sum(partials) / jnp.float32(_B * _N * _K)
